# baseline (speedup 1.0000x reference)
import functools

import jax
import jax.numpy as jnp
from jax import lax
from jax.experimental import pallas as pl
from jax.experimental.pallas import tpu as pltpu
from jax.experimental.pallas import tpu_sc as plsc

N = 10000
E_RAW = 320000
E = E_RAW + N
D_IN = 128
HDF = 16
D1 = 4 * HDF
D2 = 8 * HDF
D3 = 4 * HDF
D_OUT = 4

NPAD = 10240
SENT = N
NW = 32
CH = 128
NCA = 164
NCB = 0
EPAD = 16 * (NCA + NCB) * CH
ROWS_PER_TILE = NPAD // 16


def _attn_tail(h, ats, atd, h_ref, asad_ref, gmax_ref):
    row2 = lax.broadcasted_iota(jnp.int32, h.shape, 0)
    h = jnp.where(row2 < N, h, 0.0)
    h_ref[...] = h
    a_s = jnp.sum(h * ats[None, :], axis=1)
    a_d = jnp.sum(h * atd[None, :], axis=1)
    g = jnp.max(a_s) + jnp.max(a_d)
    g = jnp.where(g > 0, g, 0.2 * g)
    gmax_ref[...] = jnp.full((8, 128), g, jnp.float32)
    ridx = lax.broadcasted_iota(jnp.int32, (2, NPAD), 1)
    asad = jnp.stack([a_s, a_d], axis=0)
    asad_ref[...] = jnp.where(ridx < N, asad, -1e30)


def _tc_first_body(x_ref, w_ref, ats_ref, atd_ref, h_ref, asad_ref, gmax_ref):
    h = jnp.dot(x_ref[...], w_ref[...], preferred_element_type=jnp.float32)
    _attn_tail(h, ats_ref[...], atd_ref[...], h_ref, asad_ref, gmax_ref)


def _tc_mid_body(p_ref, sp_ref, b_ref, w_ref, ats_ref, atd_ref,
                 ha_ref, hb_ref, asad_ref, gmax_ref):
    s = jnp.sum(sp_ref[...], axis=0)
    agg = p_ref[0] + p_ref[1]
    hin = agg / s[:, None] + b_ref[...][None, :]
    hin = hin * jax.nn.sigmoid(hin)
    row2 = lax.broadcasted_iota(jnp.int32, hin.shape, 0)
    hin = jnp.where(row2 < N, hin, 0.0)
    h = jnp.dot(hin, w_ref[...], preferred_element_type=jnp.float32)
    row2 = lax.broadcasted_iota(jnp.int32, h.shape, 0)
    h = jnp.where(row2 < N, h, 0.0)
    ha_ref[...] = h[:, :D1]
    hb_ref[...] = h[:, D1:]
    ats, atd = ats_ref[...], atd_ref[...]
    a_s = jnp.sum(h * ats[None, :], axis=1)
    a_d = jnp.sum(h * atd[None, :], axis=1)
    g = jnp.max(a_s) + jnp.max(a_d)
    g = jnp.where(g > 0, g, 0.2 * g)
    gmax_ref[...] = jnp.full((8, 128), g, jnp.float32)
    ridx = lax.broadcasted_iota(jnp.int32, (2, NPAD), 1)
    asad = jnp.stack([a_s, a_d], axis=0)
    asad_ref[...] = jnp.where(ridx < N, asad, -1e30)


def _tc_mid2_body(pa_ref, pb_ref, sp_ref, b_ref, w_ref, ats_ref, atd_ref,
                  h_ref, asad_ref, gmax_ref):
    s = jnp.sum(sp_ref[...], axis=0)
    agg = jnp.concatenate([pa_ref[0] + pa_ref[1], pb_ref[0] + pb_ref[1]],
                          axis=1)
    hin = agg / s[:, None] + b_ref[...][None, :]
    hin = hin * jax.nn.sigmoid(hin)
    row2 = lax.broadcasted_iota(jnp.int32, hin.shape, 0)
    hin = jnp.where(row2 < N, hin, 0.0)
    h = jnp.dot(hin, w_ref[...], preferred_element_type=jnp.float32)
    _attn_tail(h, ats_ref[...], atd_ref[...], h_ref, asad_ref, gmax_ref)


def _tc_fin_body(p_ref, sp_ref, b3_ref, w5_ref, b5_ref, wo_ref, bo_ref,
                 out_ref):
    s = jnp.sum(sp_ref[...], axis=0)
    agg = p_ref[0] + p_ref[1]
    h3 = agg / s[:, None] + b3_ref[...][None, :]
    h3 = h3 * jax.nn.sigmoid(h3)
    row2 = lax.broadcasted_iota(jnp.int32, h3.shape, 0)
    h3 = jnp.where(row2 < N, h3, 0.0)
    h4 = jnp.dot(h3, w5_ref[...], preferred_element_type=jnp.float32)
    h4 = h4 + b5_ref[...][None, :]
    h4 = h4 * jax.nn.sigmoid(h4)
    out = jnp.dot(h4, wo_ref[...], preferred_element_type=jnp.float32)
    out = out + bo_ref[...][None, :]
    out_ref[...] = out * jax.nn.sigmoid(out)


def _tc_first(x_pad, w, ats, atd, dout):
    return pl.pallas_call(
        _tc_first_body,
        out_shape=[
            jax.ShapeDtypeStruct((NPAD, dout), jnp.float32),
            jax.ShapeDtypeStruct((2, NPAD), jnp.float32),
            jax.ShapeDtypeStruct((8, 128), jnp.float32),
        ],
    )(x_pad, w, ats, atd)


def _tc_mid(p, sp, b, w, ats, atd):
    return pl.pallas_call(
        _tc_mid_body,
        out_shape=[
            jax.ShapeDtypeStruct((NPAD, D1), jnp.float32),
            jax.ShapeDtypeStruct((NPAD, D1), jnp.float32),
            jax.ShapeDtypeStruct((2, NPAD), jnp.float32),
            jax.ShapeDtypeStruct((8, 128), jnp.float32),
        ],
    )(p, sp, b, w, ats, atd)


def _tc_mid2(pa, pb, sp, b, w, ats, atd, dout):
    return pl.pallas_call(
        _tc_mid2_body,
        out_shape=[
            jax.ShapeDtypeStruct((NPAD, dout), jnp.float32),
            jax.ShapeDtypeStruct((2, NPAD), jnp.float32),
            jax.ShapeDtypeStruct((8, 128), jnp.float32),
        ],
    )(pa, pb, sp, b, w, ats, atd)


def _tc_fin(p, sp, b3, w5, b5, wo_pad, bo_pad):
    return pl.pallas_call(
        _tc_fin_body,
        out_shape=jax.ShapeDtypeStruct((NPAD, 128), jnp.float32),
    )(p, sp, b3, w5, b5, wo_pad, bo_pad)


@functools.lru_cache(maxsize=None)
def _sc_edge_phase(d):
    mesh = plsc.VectorSubcoreMesh(core_axis_name="c", subcore_axis_name="s",
                                  num_cores=2, num_subcores=16)

    @functools.partial(
        pl.kernel,
        out_type=[
            jax.ShapeDtypeStruct((2, NPAD, d), jnp.float32),
            jax.ShapeDtypeStruct((NW, NPAD), jnp.float32),
        ],
        mesh=mesh,
        compiler_params=pltpu.CompilerParams(needs_layout_passes=False,
                                             use_tc_tiling_on_sc=False),
        scratch_types=[
            pltpu.VMEM((NPAD,), jnp.float32),
            pltpu.VMEM((NPAD,), jnp.float32),
            pltpu.VMEM((NCA, CH), jnp.int32),
            pltpu.VMEM((NPAD,), jnp.float32),
            pltpu.VMEM((CH,), jnp.float32),
            pltpu.VMEM((2, CH, d), jnp.float32),
            pltpu.VMEM((2, CH, d), jnp.float32),
            pltpu.VMEM((2, CH), jnp.int32),
            pltpu.VMEM((2, CH), jnp.int32),
            pltpu.VMEM((16,), jnp.float32),
            pltpu.VMEM_SHARED((NPAD, d), jnp.float32),
            pltpu.SemaphoreType.DMA,
            pltpu.SemaphoreType.DMA,
            pltpu.SemaphoreType.DMA,
            pltpu.SemaphoreType.DMA,
        ],
    )
    def edge_kernel(h_hbm, asad_hbm, gmax_hbm, pk_hbm,
                    p_hbm, sp_hbm,
                    as_v, ad_v, pk_v, sloc_v, exc_v, gbuf, sbuf, six_v, dix_v,
                    gmax_v, acc_sh, sem_g0, sem_g1, sem_s0, sem_s1):
        cid = lax.axis_index("c")
        sid = lax.axis_index("s")
        wid = cid * 16 + sid
        nch = jnp.where(cid == 0, NCA, NCB)
        sems_g = (sem_g0, sem_g1)
        sems_s = (sem_s0, sem_s1)

        def unpack_src(c, b):
            @plsc.parallel_loop(0, CH // 16, 1, unroll=8)
            def grp(g):
                pk = pk_v[c, pl.ds(g * 16, 16)]
                six_v[b, pl.ds(g * 16, 16)] = lax.shift_right_logical(pk, 14)

        def unpack_dst(c, b):
            @plsc.parallel_loop(0, CH // 16, 1, unroll=8)
            def grp(g):
                pk = pk_v[c, pl.ds(g * 16, 16)]
                dix_v[b, pl.ds(g * 16, 16)] = lax.bitwise_and(pk, 16383)

        pltpu.sync_copy(pk_hbm.at[wid], pk_v)
        for b in range(2):
            @pl.when(b < nch)
            def _():
                unpack_src(b, b)
                pltpu.async_copy(h_hbm.at[six_v.at[b]], gbuf.at[b], sems_g[b])
        pltpu.sync_copy(asad_hbm.at[0], as_v)
        pltpu.sync_copy(asad_hbm.at[1], ad_v)
        pltpu.sync_copy(gmax_hbm.at[0, pl.ds(0, 16)], gmax_v)

        zeros16 = jnp.zeros((16,), jnp.float32)

        def zero_rows(i, _):
            for j in range(d // 16):
                sbuf[0, i, pl.ds(j * 16, 16)] = zeros16
            return 0

        lax.fori_loop(0, CH, zero_rows, 0)

        def zero_s(i, _):
            sloc_v[pl.ds(i * 16, 16)] = zeros16
            return 0

        lax.fori_loop(0, NPAD // 16, zero_s, 0)

        base = sid * ROWS_PER_TILE
        sb0 = sbuf.at[0]
        for k in range(ROWS_PER_TILE // CH):
            pltpu.sync_copy(sb0, acc_sh.at[pl.ds(base + k * CH, CH)])
        plsc.subcore_barrier()

        gmax = gmax_v[...]

        sc_ring = jax.named_scope("sc_ring")
        sc_ring.__enter__()

        def ring(cc, _):
            for b in range(2):
                c = cc * 2 + b
                gb = gbuf.at[b]
                sb = sbuf.at[b]
                pltpu.make_async_copy(h_hbm.at[six_v.at[b]], gb,
                                      sems_g[b]).wait()

                @pl.when(cc > 0)
                def _():
                    pltpu.make_async_copy(sb, acc_sh.at[dix_v.at[b]],
                                          sems_s[b]).wait()

                unpack_dst(c, b)

                @plsc.parallel_loop(0, CH // 16, 1, unroll=2)
                def grp(g):
                    s16 = six_v[b, pl.ds(g * 16, 16)]
                    d16 = dix_v[b, pl.ds(g * 16, 16)]
                    va = plsc.load_gather(as_v, [s16])
                    vd = plsc.load_gather(ad_v, [d16])
                    e = va + vd
                    e = jnp.where(e > 0, e, 0.2 * e)
                    ex = jnp.exp(e - gmax)
                    exc_v[pl.ds(g * 16, 16)] = ex
                    plsc.addupdate_scatter(sloc_v, [d16], ex)

                @plsc.parallel_loop(0, CH, 1, unroll=4)
                def scale(i):
                    w = plsc.load_gather(exc_v, [jnp.full((16,), i, jnp.int32)])
                    for j in range(d // 16):
                        sb[i, pl.ds(j * 16, 16)] = gb[i, pl.ds(j * 16, 16)] * w

                pltpu.async_copy(sb, acc_sh.at[dix_v.at[b]], sems_s[b],
                                 add=True)

                @pl.when(c + 2 < nch)
                def _():
                    unpack_src(c + 2, b)
                    pltpu.async_copy(h_hbm.at[six_v.at[b]], gb, sems_g[b])

            return 0

        lax.fori_loop(0, nch // 2, ring, 0)
        for b in range(2):
            @pl.when(b < nch)
            def _():
                pltpu.make_async_copy(sbuf.at[b], acc_sh.at[dix_v.at[b]],
                                      sems_s[b]).wait()
        pltpu.sync_copy(sloc_v, sp_hbm.at[wid])
        plsc.subcore_barrier()
        sc_ring.__exit__(None, None, None)

        with jax.named_scope("sc_drain"):
            pltpu.sync_copy(acc_sh.at[pl.ds(base, ROWS_PER_TILE)],
                            p_hbm.at[cid, pl.ds(base, ROWS_PER_TILE)])

    return edge_kernel


def _sc64(*args):
    return _sc_edge_phase(D1)(*args)


def kernel(x, edge_index, W1, as1, ad1, b1, W2, as2, ad2, b2,
           W3, as3, ad3, b3, W5, b5, Wo, bo):
    loops = jnp.arange(N, dtype=jnp.int32)
    padi = jnp.full((EPAD - E,), SENT, jnp.int32)
    src = jnp.concatenate([edge_index[0].astype(jnp.int32), loops, padi])
    dst = jnp.concatenate([edge_index[1].astype(jnp.int32), loops, padi])
    flat = (src << 14) | dst
    sentp = (SENT << 14) | SENT
    na = 16 * NCA * CH
    pk_a = flat[:na].reshape(16, NCA, CH)
    pk_b = flat[na:].reshape(16, NCB, CH)
    pk_b = jnp.concatenate(
        [pk_b, jnp.full((16, NCA - NCB, CH), sentp, jnp.int32)], axis=1)
    packed = jnp.concatenate([pk_a, pk_b], axis=0)

    x_pad = jnp.zeros((NPAD, D_IN), jnp.float32).at[:N].set(x)
    wo_pad = jnp.zeros((HDF, 128), jnp.float32).at[:, :D_OUT].set(Wo)
    bo_pad = jnp.zeros((128,), jnp.float32).at[:D_OUT].set(bo)

    h1, asad1, g1 = _tc_first(x_pad, W1, as1, ad1, D1)
    p1, sp1 = _sc64(h1, asad1, g1, packed)
    h2a, h2b, asad2, g2 = _tc_mid(p1, sp1, b1, W2, as2, ad2)
    p2a, sp2 = _sc64(h2a, asad2, g2, packed)
    p2b, _sp2b = _sc64(h2b, asad2, g2, packed)
    h3, asad3, g3 = _tc_mid2(p2a, p2b, sp2, b2, W3, as3, ad3, D3)
    p3, sp3 = _sc64(h3, asad3, g3, packed)
    out = _tc_fin(p3, sp3, b3, W5, b5, wo_pad, bo_pad)
    return out[:N, :D_OUT]


# speedup vs baseline: 2.1156x; 2.1156x over previous
import functools

import jax
import jax.numpy as jnp
from jax import lax
from jax.experimental import pallas as pl
from jax.experimental.pallas import tpu as pltpu
from jax.experimental.pallas import tpu_sc as plsc

N = 10000
E_RAW = 320000
E = E_RAW + N
D_IN = 128
HDF = 16
D1 = 4 * HDF
D2 = 8 * HDF
D3 = 4 * HDF
D_OUT = 4

NPAD = 10240
SENT = N
NW = 32
CH = 128
NCA = 132
NCB = 32
EPAD = 16 * (NCA + NCB) * CH
ROWS_PER_TILE = NPAD // 16


import numpy as _np


def _perm(dim):
    p = []
    for o in range(0, dim, 32):
        p.extend(o + 2 * k for k in range(16))
        p.extend(o + 2 * k + 1 for k in range(16))
    return _np.array(p)


_P64 = _perm(64)
_P128 = _perm(128)


def _attn_tail(h, ats, atd, h_ref, asad_ref, gmax_ref):
    row2 = lax.broadcasted_iota(jnp.int32, h.shape, 0)
    h = jnp.where(row2 < N, h, 0.0)
    h_ref[...] = h.astype(jnp.bfloat16)
    a_s = jnp.sum(h * ats[None, :], axis=1)
    a_d = jnp.sum(h * atd[None, :], axis=1)
    g = jnp.max(a_s) + jnp.max(a_d)
    g = jnp.where(g > 0, g, 0.2 * g)
    gmax_ref[...] = jnp.full((8, 128), g, jnp.float32)
    ridx = lax.broadcasted_iota(jnp.int32, (2, NPAD), 1)
    asad = jnp.stack([a_s, a_d], axis=0)
    asad_ref[...] = jnp.where(ridx < N, asad, -1e30)


def _tc_first_body(x_ref, w_ref, ats_ref, atd_ref, h_ref, asad_ref, gmax_ref):
    h = jnp.dot(x_ref[...], w_ref[...], preferred_element_type=jnp.float32)
    _attn_tail(h, ats_ref[...], atd_ref[...], h_ref, asad_ref, gmax_ref)


def _tc_mid_body(p_ref, sp_ref, b_ref, w_ref, ats_ref, atd_ref,
                 ha_ref, hb_ref, asad_ref, gmax_ref):
    s = jnp.sum(sp_ref[...], axis=0)
    agg = p_ref[0] + p_ref[1]
    hin = agg / s[:, None] + b_ref[...][None, :]
    hin = hin * jax.nn.sigmoid(hin)
    row2 = lax.broadcasted_iota(jnp.int32, hin.shape, 0)
    hin = jnp.where(row2 < N, hin, 0.0)
    h = jnp.dot(hin, w_ref[...], preferred_element_type=jnp.float32)
    row2 = lax.broadcasted_iota(jnp.int32, h.shape, 0)
    h = jnp.where(row2 < N, h, 0.0)
    ha_ref[...] = h[:, :D1].astype(jnp.bfloat16)
    hb_ref[...] = h[:, D1:].astype(jnp.bfloat16)
    ats, atd = ats_ref[...], atd_ref[...]
    a_s = jnp.sum(h * ats[None, :], axis=1)
    a_d = jnp.sum(h * atd[None, :], axis=1)
    g = jnp.max(a_s) + jnp.max(a_d)
    g = jnp.where(g > 0, g, 0.2 * g)
    gmax_ref[...] = jnp.full((8, 128), g, jnp.float32)
    ridx = lax.broadcasted_iota(jnp.int32, (2, NPAD), 1)
    asad = jnp.stack([a_s, a_d], axis=0)
    asad_ref[...] = jnp.where(ridx < N, asad, -1e30)


def _tc_mid2_body(pa_ref, pb_ref, sp_ref, b_ref, w_ref, ats_ref, atd_ref,
                  h_ref, asad_ref, gmax_ref):
    s = jnp.sum(sp_ref[...], axis=0)
    agg = jnp.concatenate([pa_ref[0] + pa_ref[1], pb_ref[0] + pb_ref[1]],
                          axis=1)
    hin = agg / s[:, None] + b_ref[...][None, :]
    hin = hin * jax.nn.sigmoid(hin)
    row2 = lax.broadcasted_iota(jnp.int32, hin.shape, 0)
    hin = jnp.where(row2 < N, hin, 0.0)
    h = jnp.dot(hin, w_ref[...], preferred_element_type=jnp.float32)
    _attn_tail(h, ats_ref[...], atd_ref[...], h_ref, asad_ref, gmax_ref)


def _tc_fin_body(p_ref, sp_ref, b3_ref, w5_ref, b5_ref, wo_ref, bo_ref,
                 out_ref):
    s = jnp.sum(sp_ref[...], axis=0)
    agg = p_ref[0] + p_ref[1]
    h3 = agg / s[:, None] + b3_ref[...][None, :]
    h3 = h3 * jax.nn.sigmoid(h3)
    row2 = lax.broadcasted_iota(jnp.int32, h3.shape, 0)
    h3 = jnp.where(row2 < N, h3, 0.0)
    h4 = jnp.dot(h3, w5_ref[...], preferred_element_type=jnp.float32)
    h4 = h4 + b5_ref[...][None, :]
    h4 = h4 * jax.nn.sigmoid(h4)
    out = jnp.dot(h4, wo_ref[...], preferred_element_type=jnp.float32)
    out = out + bo_ref[...][None, :]
    out_ref[...] = out * jax.nn.sigmoid(out)


def _tc_first(x_pad, w, ats, atd, dout):
    return pl.pallas_call(
        _tc_first_body,
        out_shape=[
            jax.ShapeDtypeStruct((NPAD, dout), jnp.bfloat16),
            jax.ShapeDtypeStruct((2, NPAD), jnp.float32),
            jax.ShapeDtypeStruct((8, 128), jnp.float32),
        ],
    )(x_pad, w, ats, atd)


def _tc_mid(p, sp, b, w, ats, atd):
    return pl.pallas_call(
        _tc_mid_body,
        out_shape=[
            jax.ShapeDtypeStruct((NPAD, D1), jnp.bfloat16),
            jax.ShapeDtypeStruct((NPAD, D1), jnp.bfloat16),
            jax.ShapeDtypeStruct((2, NPAD), jnp.float32),
            jax.ShapeDtypeStruct((8, 128), jnp.float32),
        ],
    )(p, sp, b, w, ats, atd)


def _tc_mid2(pa, pb, sp, b, w, ats, atd, dout):
    return pl.pallas_call(
        _tc_mid2_body,
        out_shape=[
            jax.ShapeDtypeStruct((NPAD, dout), jnp.bfloat16),
            jax.ShapeDtypeStruct((2, NPAD), jnp.float32),
            jax.ShapeDtypeStruct((8, 128), jnp.float32),
        ],
    )(pa, pb, sp, b, w, ats, atd)


def _tc_fin(p, sp, b3, w5, b5, wo_pad, bo_pad):
    return pl.pallas_call(
        _tc_fin_body,
        out_shape=jax.ShapeDtypeStruct((NPAD, 128), jnp.float32),
    )(p, sp, b3, w5, b5, wo_pad, bo_pad)


@functools.lru_cache(maxsize=None)
def _sc_edge_phase(d):
    mesh = plsc.VectorSubcoreMesh(core_axis_name="c", subcore_axis_name="s",
                                  num_cores=2, num_subcores=16)

    @functools.partial(
        pl.kernel,
        out_type=[
            jax.ShapeDtypeStruct((2, NPAD, d), jnp.float32),
            jax.ShapeDtypeStruct((NW, NPAD), jnp.float32),
        ],
        mesh=mesh,
        compiler_params=pltpu.CompilerParams(needs_layout_passes=False,
                                             use_tc_tiling_on_sc=False),
        scratch_types=[
            pltpu.VMEM((NPAD,), jnp.float32),
            pltpu.VMEM((NPAD,), jnp.float32),
            pltpu.VMEM((NCA, CH), jnp.int32),
            pltpu.VMEM((NPAD,), jnp.float32),
            pltpu.VMEM((CH,), jnp.float32),
            pltpu.VMEM((2, CH, d), jnp.bfloat16),
            pltpu.VMEM((2, CH, d), jnp.float32),
            pltpu.VMEM((2, CH), jnp.int32),
            pltpu.VMEM((2, CH), jnp.int32),
            pltpu.VMEM((16,), jnp.float32),
            pltpu.VMEM_SHARED((NPAD, d), jnp.float32),
            pltpu.SemaphoreType.DMA,
            pltpu.SemaphoreType.DMA,
            pltpu.SemaphoreType.DMA,
            pltpu.SemaphoreType.DMA,
        ],
    )
    def edge_kernel(h_hbm, asad_hbm, gmax_hbm, pk_hbm,
                    p_hbm, sp_hbm,
                    as_v, ad_v, pk_v, sloc_v, exc_v, gbuf, sbuf, six_v, dix_v,
                    gmax_v, acc_sh, sem_g0, sem_g1, sem_s0, sem_s1):
        cid = lax.axis_index("c")
        sid = lax.axis_index("s")
        wid = cid * 16 + sid
        nch = jnp.where(cid == 0, NCA, NCB)
        sems_g = (sem_g0, sem_g1)
        sems_s = (sem_s0, sem_s1)

        def unpack_src(c, b):
            @plsc.parallel_loop(0, CH // 16, 1, unroll=8)
            def grp(g):
                pk = pk_v[c, pl.ds(g * 16, 16)]
                six_v[b, pl.ds(g * 16, 16)] = lax.shift_right_logical(pk, 14)

        def unpack_dst(c, b):
            @plsc.parallel_loop(0, CH // 16, 1, unroll=8)
            def grp(g):
                pk = pk_v[c, pl.ds(g * 16, 16)]
                dix_v[b, pl.ds(g * 16, 16)] = lax.bitwise_and(pk, 16383)

        pltpu.sync_copy(pk_hbm.at[wid], pk_v)
        for b in range(2):
            @pl.when(b < nch)
            def _():
                unpack_src(b, b)
                pltpu.async_copy(h_hbm.at[six_v.at[b]], gbuf.at[b], sems_g[b])
        pltpu.sync_copy(asad_hbm.at[0], as_v)
        pltpu.sync_copy(asad_hbm.at[1], ad_v)
        pltpu.sync_copy(gmax_hbm.at[0, pl.ds(0, 16)], gmax_v)

        zeros16 = jnp.zeros((16,), jnp.float32)

        def zero_rows(i, _):
            for j in range(d // 16):
                sbuf[0, i, pl.ds(j * 16, 16)] = zeros16
            return 0

        lax.fori_loop(0, CH, zero_rows, 0)

        def zero_s(i, _):
            sloc_v[pl.ds(i * 16, 16)] = zeros16
            return 0

        lax.fori_loop(0, NPAD // 16, zero_s, 0)

        base = sid * ROWS_PER_TILE
        sb0 = sbuf.at[0]
        for k in range(ROWS_PER_TILE // CH):
            pltpu.sync_copy(sb0, acc_sh.at[pl.ds(base + k * CH, CH)])
        plsc.subcore_barrier()

        gmax = gmax_v[...]

        sc_ring = jax.named_scope("sc_ring")
        sc_ring.__enter__()

        def ring(cc, _):
            for b in range(2):
                c = cc * 2 + b
                gb = gbuf.at[b]
                sb = sbuf.at[b]
                pltpu.make_async_copy(h_hbm.at[six_v.at[b]], gb,
                                      sems_g[b]).wait()

                @pl.when(cc > 0)
                def _():
                    pltpu.make_async_copy(sb, acc_sh.at[dix_v.at[b]],
                                          sems_s[b]).wait()

                unpack_dst(c, b)

                @plsc.parallel_loop(0, CH // 16, 1, unroll=2)
                def grp(g):
                    s16 = six_v[b, pl.ds(g * 16, 16)]
                    d16 = dix_v[b, pl.ds(g * 16, 16)]
                    va = plsc.load_gather(as_v, [s16])
                    vd = plsc.load_gather(ad_v, [d16])
                    e = va + vd
                    e = jnp.where(e > 0, e, 0.2 * e)
                    ex = jnp.exp(e - gmax)
                    exc_v[pl.ds(g * 16, 16)] = ex
                    plsc.addupdate_scatter(sloc_v, [d16], ex)

                @plsc.parallel_loop(0, CH, 1, unroll=4)
                def scale(i):
                    w = plsc.load_gather(exc_v, [jnp.full((16,), i, jnp.int32)])
                    for j in range(d // 32):
                        x = gb[i, pl.ds(j * 32, 32)]
                        lo, hi = plsc.unpack(
                            x, format=plsc.PackFormat.INTERLEAVED)
                        sb[i, pl.ds(j * 32, 16)] = lo * w
                        sb[i, pl.ds(j * 32 + 16, 16)] = hi * w

                pltpu.async_copy(sb, acc_sh.at[dix_v.at[b]], sems_s[b],
                                 add=True)

                @pl.when(c + 2 < nch)
                def _():
                    unpack_src(c + 2, b)
                    pltpu.async_copy(h_hbm.at[six_v.at[b]], gb, sems_g[b])

            return 0

        lax.fori_loop(0, nch // 2, ring, 0)
        for b in range(2):
            @pl.when(b < nch)
            def _():
                pltpu.make_async_copy(sbuf.at[b], acc_sh.at[dix_v.at[b]],
                                      sems_s[b]).wait()
        pltpu.sync_copy(sloc_v, sp_hbm.at[wid])
        plsc.subcore_barrier()
        sc_ring.__exit__(None, None, None)

        with jax.named_scope("sc_drain"):
            pltpu.sync_copy(acc_sh.at[pl.ds(base, ROWS_PER_TILE)],
                            p_hbm.at[cid, pl.ds(base, ROWS_PER_TILE)])

    return edge_kernel


def _sc64(*args):
    return _sc_edge_phase(D1)(*args)


def kernel(x, edge_index, W1, as1, ad1, b1, W2, as2, ad2, b2,
           W3, as3, ad3, b3, W5, b5, Wo, bo):
    loops = jnp.arange(N, dtype=jnp.int32)
    padi = jnp.full((EPAD - E,), SENT, jnp.int32)
    src = jnp.concatenate([edge_index[0].astype(jnp.int32), loops, padi])
    dst = jnp.concatenate([edge_index[1].astype(jnp.int32), loops, padi])
    flat = (src << 14) | dst
    sentp = (SENT << 14) | SENT
    na = 16 * NCA * CH
    pk_a = flat[:na].reshape(16, NCA, CH)
    pk_b = flat[na:].reshape(16, NCB, CH)
    pk_b = jnp.concatenate(
        [pk_b, jnp.full((16, NCA - NCB, CH), sentp, jnp.int32)], axis=1)
    packed = jnp.concatenate([pk_a, pk_b], axis=0)

    x_pad = jnp.zeros((NPAD, D_IN), jnp.float32).at[:N].set(x)
    wo_pad = jnp.zeros((HDF, 128), jnp.float32).at[:, :D_OUT].set(Wo)
    bo_pad = jnp.zeros((128,), jnp.float32).at[:D_OUT].set(bo)

    b1p, w2p = b1[_P64], W2[_P64, :]
    b2p, w3p = b2[_P128], W3[_P128, :]
    b3p, w5p = b3[_P64], W5[_P64, :]

    h1, asad1, g1 = _tc_first(x_pad, W1, as1, ad1, D1)
    p1, sp1 = _sc64(h1, asad1, g1, packed)
    h2a, h2b, asad2, g2 = _tc_mid(p1, sp1, b1p, w2p, as2, ad2)
    p2a, sp2 = _sc64(h2a, asad2, g2, packed)
    p2b, _sp2b = _sc64(h2b, asad2, g2, packed)
    h3, asad3, g3 = _tc_mid2(p2a, p2b, sp2, b2p, w3p, as3, ad3, D3)
    p3, sp3 = _sc64(h3, asad3, g3, packed)
    out = _tc_fin(p3, sp3, b3p, w5p, b5, wo_pad, bo_pad)
    return out[:N, :D_OUT]


# speedup vs baseline: 2.1861x; 1.0333x over previous
import functools

import jax
import jax.numpy as jnp
from jax import lax
from jax.experimental import pallas as pl
from jax.experimental.pallas import tpu as pltpu
from jax.experimental.pallas import tpu_sc as plsc

N = 10000
E_RAW = 320000
E = E_RAW + N
D_IN = 128
HDF = 16
D1 = 4 * HDF
D2 = 8 * HDF
D3 = 4 * HDF
D_OUT = 4

NPAD = 10240
SENT = N
NW = 32
CH = 128
NCA = 124
NCB = 40
EPAD = 16 * (NCA + NCB) * CH
ROWS_PER_TILE = NPAD // 16


import numpy as _np


def _perm(dim):
    p = []
    for o in range(0, dim, 32):
        p.extend(o + 2 * k for k in range(16))
        p.extend(o + 2 * k + 1 for k in range(16))
    return _np.array(p)


_P64 = _perm(64)
_P128 = _perm(128)


def _attn_tail(h, ats, atd, h_ref, asad_ref, gmax_ref):
    row2 = lax.broadcasted_iota(jnp.int32, h.shape, 0)
    h = jnp.where(row2 < N, h, 0.0)
    h_ref[...] = h.astype(jnp.bfloat16)
    a_s = jnp.sum(h * ats[None, :], axis=1)
    a_d = jnp.sum(h * atd[None, :], axis=1)
    g = jnp.max(a_s) + jnp.max(a_d)
    g = jnp.where(g > 0, g, 0.2 * g)
    gmax_ref[...] = jnp.full((8, 128), g, jnp.float32)
    ridx = lax.broadcasted_iota(jnp.int32, (2, NPAD), 1)
    asad = jnp.stack([a_s, a_d], axis=0)
    asad_ref[...] = jnp.where(ridx < N, asad, -1e30)


def _tc_first_body(x_ref, w_ref, ats_ref, atd_ref, h_ref, asad_ref, gmax_ref):
    h = jnp.dot(x_ref[...], w_ref[...], preferred_element_type=jnp.float32)
    _attn_tail(h, ats_ref[...], atd_ref[...], h_ref, asad_ref, gmax_ref)


def _tc_mid_body(p_ref, sp_ref, b_ref, w_ref, ats_ref, atd_ref,
                 ha_ref, hb_ref, asad_ref, gmax_ref):
    s = jnp.sum(sp_ref[...], axis=0)
    agg = p_ref[0] + p_ref[1]
    hin = agg / s[:, None] + b_ref[...][None, :]
    hin = hin * jax.nn.sigmoid(hin)
    row2 = lax.broadcasted_iota(jnp.int32, hin.shape, 0)
    hin = jnp.where(row2 < N, hin, 0.0)
    h = jnp.dot(hin, w_ref[...], preferred_element_type=jnp.float32)
    row2 = lax.broadcasted_iota(jnp.int32, h.shape, 0)
    h = jnp.where(row2 < N, h, 0.0)
    ha_ref[...] = h[:, :D1].astype(jnp.bfloat16)
    hb_ref[...] = h[:, D1:].astype(jnp.bfloat16)
    ats, atd = ats_ref[...], atd_ref[...]
    a_s = jnp.sum(h * ats[None, :], axis=1)
    a_d = jnp.sum(h * atd[None, :], axis=1)
    g = jnp.max(a_s) + jnp.max(a_d)
    g = jnp.where(g > 0, g, 0.2 * g)
    gmax_ref[...] = jnp.full((8, 128), g, jnp.float32)
    ridx = lax.broadcasted_iota(jnp.int32, (2, NPAD), 1)
    asad = jnp.stack([a_s, a_d], axis=0)
    asad_ref[...] = jnp.where(ridx < N, asad, -1e30)


def _tc_mid2_body(pa_ref, pb_ref, sp_ref, b_ref, w_ref, ats_ref, atd_ref,
                  h_ref, asad_ref, gmax_ref):
    s = jnp.sum(sp_ref[...], axis=0)
    agg = jnp.concatenate([pa_ref[0] + pa_ref[1], pb_ref[0] + pb_ref[1]],
                          axis=1)
    hin = agg / s[:, None] + b_ref[...][None, :]
    hin = hin * jax.nn.sigmoid(hin)
    row2 = lax.broadcasted_iota(jnp.int32, hin.shape, 0)
    hin = jnp.where(row2 < N, hin, 0.0)
    h = jnp.dot(hin, w_ref[...], preferred_element_type=jnp.float32)
    _attn_tail(h, ats_ref[...], atd_ref[...], h_ref, asad_ref, gmax_ref)


def _tc_fin_body(p_ref, sp_ref, b3_ref, w5_ref, b5_ref, wo_ref, bo_ref,
                 out_ref):
    s = jnp.sum(sp_ref[...], axis=0)
    agg = p_ref[0] + p_ref[1]
    h3 = agg / s[:, None] + b3_ref[...][None, :]
    h3 = h3 * jax.nn.sigmoid(h3)
    row2 = lax.broadcasted_iota(jnp.int32, h3.shape, 0)
    h3 = jnp.where(row2 < N, h3, 0.0)
    h4 = jnp.dot(h3, w5_ref[...], preferred_element_type=jnp.float32)
    h4 = h4 + b5_ref[...][None, :]
    h4 = h4 * jax.nn.sigmoid(h4)
    out = jnp.dot(h4, wo_ref[...], preferred_element_type=jnp.float32)
    out = out + bo_ref[...][None, :]
    out_ref[...] = out * jax.nn.sigmoid(out)


def _tc_first(x_pad, w, ats, atd, dout):
    return pl.pallas_call(
        _tc_first_body,
        out_shape=[
            jax.ShapeDtypeStruct((NPAD, dout), jnp.bfloat16),
            jax.ShapeDtypeStruct((2, NPAD), jnp.float32),
            jax.ShapeDtypeStruct((8, 128), jnp.float32),
        ],
    )(x_pad, w, ats, atd)


def _tc_mid(p, sp, b, w, ats, atd):
    return pl.pallas_call(
        _tc_mid_body,
        out_shape=[
            jax.ShapeDtypeStruct((NPAD, D1), jnp.bfloat16),
            jax.ShapeDtypeStruct((NPAD, D1), jnp.bfloat16),
            jax.ShapeDtypeStruct((2, NPAD), jnp.float32),
            jax.ShapeDtypeStruct((8, 128), jnp.float32),
        ],
    )(p, sp, b, w, ats, atd)


def _tc_mid2(pa, pb, sp, b, w, ats, atd, dout):
    return pl.pallas_call(
        _tc_mid2_body,
        out_shape=[
            jax.ShapeDtypeStruct((NPAD, dout), jnp.bfloat16),
            jax.ShapeDtypeStruct((2, NPAD), jnp.float32),
            jax.ShapeDtypeStruct((8, 128), jnp.float32),
        ],
    )(pa, pb, sp, b, w, ats, atd)


def _tc_fin(p, sp, b3, w5, b5, wo_pad, bo_pad):
    return pl.pallas_call(
        _tc_fin_body,
        out_shape=jax.ShapeDtypeStruct((NPAD, 128), jnp.float32),
    )(p, sp, b3, w5, b5, wo_pad, bo_pad)


@functools.lru_cache(maxsize=None)
def _sc_edge_phase(d):
    mesh = plsc.VectorSubcoreMesh(core_axis_name="c", subcore_axis_name="s",
                                  num_cores=2, num_subcores=16)

    @functools.partial(
        pl.kernel,
        out_type=[
            jax.ShapeDtypeStruct((2, NPAD, d), jnp.float32),
            jax.ShapeDtypeStruct((NW, NPAD), jnp.float32),
        ],
        mesh=mesh,
        compiler_params=pltpu.CompilerParams(needs_layout_passes=False,
                                             use_tc_tiling_on_sc=False),
        scratch_types=[
            pltpu.VMEM((NPAD,), jnp.float32),
            pltpu.VMEM((NPAD,), jnp.float32),
            pltpu.VMEM((NCA, CH), jnp.int32),
            pltpu.VMEM((NPAD,), jnp.float32),
            pltpu.VMEM((CH,), jnp.float32),
            pltpu.VMEM((2, CH, d), jnp.bfloat16),
            pltpu.VMEM((2, CH, d), jnp.float32),
            pltpu.VMEM((2, CH), jnp.int32),
            pltpu.VMEM((2, CH), jnp.int32),
            pltpu.VMEM((16,), jnp.float32),
            pltpu.VMEM_SHARED((NPAD, d), jnp.float32),
            pltpu.SemaphoreType.DMA,
            pltpu.SemaphoreType.DMA,
            pltpu.SemaphoreType.DMA,
            pltpu.SemaphoreType.DMA,
        ],
    )
    def edge_kernel(h_hbm, asad_hbm, gmax_hbm, pk_hbm,
                    p_hbm, sp_hbm,
                    as_v, ad_v, pk_v, sloc_v, exc_v, gbuf, sbuf, six_v, dix_v,
                    gmax_v, acc_sh, sem_g0, sem_g1, sem_s0, sem_s1):
        cid = lax.axis_index("c")
        sid = lax.axis_index("s")
        wid = cid * 16 + sid
        nch = jnp.where(cid == 0, NCA, NCB)
        sems_g = (sem_g0, sem_g1)
        sems_s = (sem_s0, sem_s1)

        def unpack_src(c, b):
            @plsc.parallel_loop(0, CH // 16, 1, unroll=8)
            def grp(g):
                pk = pk_v[c, pl.ds(g * 16, 16)]
                six_v[b, pl.ds(g * 16, 16)] = lax.shift_right_logical(pk, 14)

        def unpack_dst(c, b):
            @plsc.parallel_loop(0, CH // 16, 1, unroll=8)
            def grp(g):
                pk = pk_v[c, pl.ds(g * 16, 16)]
                dix_v[b, pl.ds(g * 16, 16)] = lax.bitwise_and(pk, 16383)

        pltpu.sync_copy(pk_hbm.at[wid], pk_v)
        for b in range(2):
            @pl.when(b < nch)
            def _():
                unpack_src(b, b)
                pltpu.async_copy(h_hbm.at[six_v.at[b]], gbuf.at[b], sems_g[b])
        pltpu.sync_copy(asad_hbm.at[0], as_v)
        pltpu.sync_copy(asad_hbm.at[1], ad_v)
        pltpu.sync_copy(gmax_hbm.at[0, pl.ds(0, 16)], gmax_v)

        zeros16 = jnp.zeros((16,), jnp.float32)

        def zero_rows(i, _):
            for j in range(d // 16):
                sbuf[0, i, pl.ds(j * 16, 16)] = zeros16
            return 0

        lax.fori_loop(0, CH, zero_rows, 0)

        def zero_s(i, _):
            sloc_v[pl.ds(i * 16, 16)] = zeros16
            return 0

        lax.fori_loop(0, NPAD // 16, zero_s, 0)

        base = sid * ROWS_PER_TILE
        sb0 = sbuf.at[0]
        for k in range(ROWS_PER_TILE // CH):
            pltpu.sync_copy(sb0, acc_sh.at[pl.ds(base + k * CH, CH)])
        plsc.subcore_barrier()

        gmax = gmax_v[...]

        sc_ring = jax.named_scope("sc_ring")
        sc_ring.__enter__()

        def ring(cc, _):
            for b in range(2):
                c = cc * 2 + b
                gb = gbuf.at[b]
                sb = sbuf.at[b]
                pltpu.make_async_copy(h_hbm.at[six_v.at[b]], gb,
                                      sems_g[b]).wait()

                @pl.when(cc > 0)
                def _():
                    pltpu.make_async_copy(sb, acc_sh.at[dix_v.at[b]],
                                          sems_s[b]).wait()

                unpack_dst(c, b)

                @plsc.parallel_loop(0, CH // 16, 1, unroll=2)
                def grp(g):
                    s16 = six_v[b, pl.ds(g * 16, 16)]
                    d16 = dix_v[b, pl.ds(g * 16, 16)]
                    va = plsc.load_gather(as_v, [s16])
                    vd = plsc.load_gather(ad_v, [d16])
                    e = va + vd
                    e = jnp.where(e > 0, e, 0.2 * e)
                    ex = jnp.exp(e - gmax)
                    exc_v[pl.ds(g * 16, 16)] = ex
                    plsc.addupdate_scatter(sloc_v, [d16], ex)

                @plsc.parallel_loop(0, CH, 1, unroll=4)
                def scale(i):
                    w = plsc.load_gather(exc_v, [jnp.full((16,), i, jnp.int32)])
                    for j in range(d // 32):
                        x = gb[i, pl.ds(j * 32, 32)]
                        lo, hi = plsc.unpack(
                            x, format=plsc.PackFormat.INTERLEAVED)
                        sb[i, pl.ds(j * 32, 16)] = lo * w
                        sb[i, pl.ds(j * 32 + 16, 16)] = hi * w

                pltpu.async_copy(sb, acc_sh.at[dix_v.at[b]], sems_s[b],
                                 add=True)

                @pl.when(c + 2 < nch)
                def _():
                    unpack_src(c + 2, b)
                    pltpu.async_copy(h_hbm.at[six_v.at[b]], gb, sems_g[b])

            return 0

        lax.fori_loop(0, nch // 2, ring, 0)
        for b in range(2):
            @pl.when(b < nch)
            def _():
                pltpu.make_async_copy(sbuf.at[b], acc_sh.at[dix_v.at[b]],
                                      sems_s[b]).wait()
        pltpu.sync_copy(sloc_v, sp_hbm.at[wid])
        plsc.subcore_barrier()
        sc_ring.__exit__(None, None, None)

        with jax.named_scope("sc_drain"):
            pltpu.sync_copy(acc_sh.at[pl.ds(base, ROWS_PER_TILE)],
                            p_hbm.at[cid, pl.ds(base, ROWS_PER_TILE)])

    return edge_kernel


def _sc64(*args):
    return _sc_edge_phase(D1)(*args)


def kernel(x, edge_index, W1, as1, ad1, b1, W2, as2, ad2, b2,
           W3, as3, ad3, b3, W5, b5, Wo, bo):
    loops = jnp.arange(N, dtype=jnp.int32)
    padi = jnp.full((EPAD - E,), SENT, jnp.int32)
    src = jnp.concatenate([edge_index[0].astype(jnp.int32), loops, padi])
    dst = jnp.concatenate([edge_index[1].astype(jnp.int32), loops, padi])
    flat = (src << 14) | dst
    sentp = (SENT << 14) | SENT
    na = 16 * NCA * CH
    pk_a = flat[:na].reshape(16, NCA, CH)
    pk_b = flat[na:].reshape(16, NCB, CH)
    pk_b = jnp.concatenate(
        [pk_b, jnp.full((16, NCA - NCB, CH), sentp, jnp.int32)], axis=1)
    packed = jnp.concatenate([pk_a, pk_b], axis=0)

    x_pad = jnp.zeros((NPAD, D_IN), jnp.float32).at[:N].set(x)
    wo_pad = jnp.zeros((HDF, 128), jnp.float32).at[:, :D_OUT].set(Wo)
    bo_pad = jnp.zeros((128,), jnp.float32).at[:D_OUT].set(bo)

    b1p, w2p = b1[_P64], W2[_P64, :]
    b2p, w3p = b2[_P128], W3[_P128, :]
    b3p, w5p = b3[_P64], W5[_P64, :]

    h1, asad1, g1 = _tc_first(x_pad, W1, as1, ad1, D1)
    p1, sp1 = _sc64(h1, asad1, g1, packed)
    h2a, h2b, asad2, g2 = _tc_mid(p1, sp1, b1p, w2p, as2, ad2)
    p2a, sp2 = _sc64(h2a, asad2, g2, packed)
    p2b, _sp2b = _sc64(h2b, asad2, g2, packed)
    h3, asad3, g3 = _tc_mid2(p2a, p2b, sp2, b2p, w3p, as3, ad3, D3)
    p3, sp3 = _sc64(h3, asad3, g3, packed)
    out = _tc_fin(p3, sp3, b3p, w5p, b5, wo_pad, bo_pad)
    return out[:N, :D_OUT]


# speedup vs baseline: 2.5973x; 1.1881x over previous
import functools

import jax
import jax.numpy as jnp
from jax import lax
from jax.experimental import pallas as pl
from jax.experimental.pallas import tpu as pltpu
from jax.experimental.pallas import tpu_sc as plsc

N = 10000
E_RAW = 320000
E = E_RAW + N
D_IN = 128
HDF = 16
D1 = 4 * HDF
D2 = 8 * HDF
D3 = 4 * HDF
D_OUT = 4

NPAD = 10240
SENT = N
NW = 32
CH = 128
NRING = 3
NCA = 123
NCB = 39
EPAD = 16 * (NCA + NCB) * CH
ROWS_PER_TILE = NPAD // 16


import numpy as _np


def _perm(dim):
    p = []
    for o in range(0, dim, 32):
        p.extend(o + 2 * k for k in range(16))
        p.extend(o + 2 * k + 1 for k in range(16))
    return _np.array(p)


_P64 = _perm(64)
_P128 = _perm(128)


def _attn_tail(h, ats, atd, h_ref, asad_ref, gmax_ref):
    row2 = lax.broadcasted_iota(jnp.int32, h.shape, 0)
    h = jnp.where(row2 < N, h, 0.0)
    h_ref[...] = h.astype(jnp.bfloat16)
    a_s = jnp.sum(h * ats[None, :], axis=1)
    a_d = jnp.sum(h * atd[None, :], axis=1)
    g = jnp.max(a_s) + jnp.max(a_d)
    g = jnp.where(g > 0, g, 0.2 * g)
    gmax_ref[...] = jnp.full((8, 128), g, jnp.float32)
    ridx = lax.broadcasted_iota(jnp.int32, (2, NPAD), 1)
    asad = jnp.stack([a_s, a_d], axis=0)
    asad_ref[...] = jnp.where(ridx < N, asad, -1e30)


def _tc_first_body(x_ref, w_ref, ats_ref, atd_ref, h_ref, asad_ref, gmax_ref):
    h = jnp.dot(x_ref[...], w_ref[...], preferred_element_type=jnp.float32)
    _attn_tail(h, ats_ref[...], atd_ref[...], h_ref, asad_ref, gmax_ref)


def _tc_mid_body(p_ref, sp_ref, b_ref, w_ref, ats_ref, atd_ref,
                 ha_ref, hb_ref, asad_ref, gmax_ref):
    s = jnp.sum(sp_ref[...], axis=0)
    agg = p_ref[0] + p_ref[1]
    hin = agg / s[:, None] + b_ref[...][None, :]
    hin = hin * jax.nn.sigmoid(hin)
    row2 = lax.broadcasted_iota(jnp.int32, hin.shape, 0)
    hin = jnp.where(row2 < N, hin, 0.0)
    h = jnp.dot(hin, w_ref[...], preferred_element_type=jnp.float32)
    row2 = lax.broadcasted_iota(jnp.int32, h.shape, 0)
    h = jnp.where(row2 < N, h, 0.0)
    ha_ref[...] = h[:, :D1].astype(jnp.bfloat16)
    hb_ref[...] = h[:, D1:].astype(jnp.bfloat16)
    ats, atd = ats_ref[...], atd_ref[...]
    a_s = jnp.sum(h * ats[None, :], axis=1)
    a_d = jnp.sum(h * atd[None, :], axis=1)
    g = jnp.max(a_s) + jnp.max(a_d)
    g = jnp.where(g > 0, g, 0.2 * g)
    gmax_ref[...] = jnp.full((8, 128), g, jnp.float32)
    ridx = lax.broadcasted_iota(jnp.int32, (2, NPAD), 1)
    asad = jnp.stack([a_s, a_d], axis=0)
    asad_ref[...] = jnp.where(ridx < N, asad, -1e30)


def _tc_mid2_body(pa_ref, pb_ref, sp_ref, b_ref, w_ref, ats_ref, atd_ref,
                  h_ref, asad_ref, gmax_ref):
    s = jnp.sum(sp_ref[...], axis=0)
    agg = jnp.concatenate([pa_ref[0] + pa_ref[1], pb_ref[0] + pb_ref[1]],
                          axis=1)
    hin = agg / s[:, None] + b_ref[...][None, :]
    hin = hin * jax.nn.sigmoid(hin)
    row2 = lax.broadcasted_iota(jnp.int32, hin.shape, 0)
    hin = jnp.where(row2 < N, hin, 0.0)
    h = jnp.dot(hin, w_ref[...], preferred_element_type=jnp.float32)
    _attn_tail(h, ats_ref[...], atd_ref[...], h_ref, asad_ref, gmax_ref)


def _tc_fin_body(p_ref, sp_ref, b3_ref, w5_ref, b5_ref, wo_ref, bo_ref,
                 out_ref):
    s = jnp.sum(sp_ref[...], axis=0)
    agg = p_ref[0] + p_ref[1]
    h3 = agg / s[:, None] + b3_ref[...][None, :]
    h3 = h3 * jax.nn.sigmoid(h3)
    row2 = lax.broadcasted_iota(jnp.int32, h3.shape, 0)
    h3 = jnp.where(row2 < N, h3, 0.0)
    h4 = jnp.dot(h3, w5_ref[...], preferred_element_type=jnp.float32)
    h4 = h4 + b5_ref[...][None, :]
    h4 = h4 * jax.nn.sigmoid(h4)
    out = jnp.dot(h4, wo_ref[...], preferred_element_type=jnp.float32)
    out = out + bo_ref[...][None, :]
    out_ref[...] = out * jax.nn.sigmoid(out)


def _tc_first(x_pad, w, ats, atd, dout):
    return pl.pallas_call(
        _tc_first_body,
        out_shape=[
            jax.ShapeDtypeStruct((NPAD, dout), jnp.bfloat16),
            jax.ShapeDtypeStruct((2, NPAD), jnp.float32),
            jax.ShapeDtypeStruct((8, 128), jnp.float32),
        ],
    )(x_pad, w, ats, atd)


def _tc_mid(p, sp, b, w, ats, atd):
    return pl.pallas_call(
        _tc_mid_body,
        out_shape=[
            jax.ShapeDtypeStruct((NPAD, D1), jnp.bfloat16),
            jax.ShapeDtypeStruct((NPAD, D1), jnp.bfloat16),
            jax.ShapeDtypeStruct((2, NPAD), jnp.float32),
            jax.ShapeDtypeStruct((8, 128), jnp.float32),
        ],
    )(p, sp, b, w, ats, atd)


def _tc_mid2(pa, pb, sp, b, w, ats, atd, dout):
    return pl.pallas_call(
        _tc_mid2_body,
        out_shape=[
            jax.ShapeDtypeStruct((NPAD, dout), jnp.bfloat16),
            jax.ShapeDtypeStruct((2, NPAD), jnp.float32),
            jax.ShapeDtypeStruct((8, 128), jnp.float32),
        ],
    )(pa, pb, sp, b, w, ats, atd)


def _tc_fin(p, sp, b3, w5, b5, wo_pad, bo_pad):
    return pl.pallas_call(
        _tc_fin_body,
        out_shape=jax.ShapeDtypeStruct((NPAD, 128), jnp.float32),
    )(p, sp, b3, w5, b5, wo_pad, bo_pad)


@functools.lru_cache(maxsize=None)
def _sc_edge_phase(d):
    mesh = plsc.VectorSubcoreMesh(core_axis_name="c", subcore_axis_name="s",
                                  num_cores=2, num_subcores=16)

    @functools.partial(
        pl.kernel,
        out_type=[
            jax.ShapeDtypeStruct((2, NPAD, d), jnp.float32),
            jax.ShapeDtypeStruct((NW, NPAD), jnp.float32),
        ],
        mesh=mesh,
        compiler_params=pltpu.CompilerParams(needs_layout_passes=False,
                                             use_tc_tiling_on_sc=False),
        scratch_types=[
            pltpu.VMEM((NPAD,), jnp.float32),
            pltpu.VMEM((NPAD,), jnp.float32),
            pltpu.VMEM((NCA, CH), jnp.int32),
            pltpu.VMEM((NPAD,), jnp.float32),
            pltpu.VMEM((CH,), jnp.float32),
            pltpu.VMEM((NRING, CH, d), jnp.bfloat16),
            pltpu.VMEM((NRING, CH, d), jnp.float32),
            pltpu.VMEM((NRING, CH), jnp.int32),
            pltpu.VMEM((NRING, CH), jnp.int32),
            pltpu.VMEM((16,), jnp.float32),
            pltpu.VMEM_SHARED((NPAD, d), jnp.float32),
            pltpu.SemaphoreType.DMA,
            pltpu.SemaphoreType.DMA,
            pltpu.SemaphoreType.DMA,
            pltpu.SemaphoreType.DMA,
            pltpu.SemaphoreType.DMA,
            pltpu.SemaphoreType.DMA,
        ],
    )
    def edge_kernel(h_hbm, asad_hbm, gmax_hbm, pk_hbm,
                    p_hbm, sp_hbm,
                    as_v, ad_v, pk_v, sloc_v, exc_v, gbuf, sbuf, six_v, dix_v,
                    gmax_v, acc_sh, sem_g0, sem_g1, sem_g2,
                    sem_s0, sem_s1, sem_s2):
        cid = lax.axis_index("c")
        sid = lax.axis_index("s")
        wid = cid * 16 + sid
        nch = jnp.where(cid == 0, NCA, NCB)
        sems_g = (sem_g0, sem_g1, sem_g2)
        sems_s = (sem_s0, sem_s1, sem_s2)

        def unpack_src(c, b):
            @plsc.parallel_loop(0, CH // 16, 1, unroll=8)
            def grp(g):
                pk = pk_v[c, pl.ds(g * 16, 16)]
                six_v[b, pl.ds(g * 16, 16)] = lax.shift_right_logical(pk, 14)

        def unpack_dst(c, b):
            @plsc.parallel_loop(0, CH // 16, 1, unroll=8)
            def grp(g):
                pk = pk_v[c, pl.ds(g * 16, 16)]
                dix_v[b, pl.ds(g * 16, 16)] = lax.bitwise_and(pk, 16383)

        pltpu.sync_copy(pk_hbm.at[wid], pk_v)
        for b in range(NRING):
            @pl.when(b < nch)
            def _():
                unpack_src(b, b)
                pltpu.async_copy(h_hbm.at[six_v.at[b]], gbuf.at[b], sems_g[b])
        pltpu.sync_copy(asad_hbm.at[0], as_v)
        pltpu.sync_copy(asad_hbm.at[1], ad_v)
        pltpu.sync_copy(gmax_hbm.at[0, pl.ds(0, 16)], gmax_v)

        zeros16 = jnp.zeros((16,), jnp.float32)

        def zero_rows(i, _):
            for j in range(d // 16):
                sbuf[0, i, pl.ds(j * 16, 16)] = zeros16
            return 0

        lax.fori_loop(0, CH, zero_rows, 0)

        def zero_s(i, _):
            sloc_v[pl.ds(i * 16, 16)] = zeros16
            return 0

        lax.fori_loop(0, NPAD // 16, zero_s, 0)

        base = sid * ROWS_PER_TILE
        sb0 = sbuf.at[0]
        for k in range(ROWS_PER_TILE // CH):
            pltpu.sync_copy(sb0, acc_sh.at[pl.ds(base + k * CH, CH)])
        plsc.subcore_barrier()

        gmax = gmax_v[...]

        sc_ring = jax.named_scope("sc_ring")
        sc_ring.__enter__()

        def ring(cc, _):
            for b in range(NRING):
                c = cc * NRING + b
                gb = gbuf.at[b]
                sb = sbuf.at[b]
                pltpu.make_async_copy(h_hbm.at[six_v.at[b]], gb,
                                      sems_g[b]).wait()

                @pl.when(cc > 0)
                def _():
                    pltpu.make_async_copy(sb, acc_sh.at[dix_v.at[b]],
                                          sems_s[b]).wait()

                unpack_dst(c, b)

                @plsc.parallel_loop(0, CH // 16, 1, unroll=2)
                def grp(g):
                    s16 = six_v[b, pl.ds(g * 16, 16)]
                    d16 = dix_v[b, pl.ds(g * 16, 16)]
                    va = plsc.load_gather(as_v, [s16])
                    vd = plsc.load_gather(ad_v, [d16])
                    e = va + vd
                    e = jnp.where(e > 0, e, 0.2 * e)
                    ex = jnp.exp(e - gmax)
                    exc_v[pl.ds(g * 16, 16)] = ex
                    plsc.addupdate_scatter(sloc_v, [d16], ex)

                @plsc.parallel_loop(0, CH, 1, unroll=4)
                def scale(i):
                    w = plsc.load_gather(exc_v, [jnp.full((16,), i, jnp.int32)])
                    for j in range(d // 32):
                        x = gb[i, pl.ds(j * 32, 32)]
                        lo, hi = plsc.unpack(
                            x, format=plsc.PackFormat.INTERLEAVED)
                        sb[i, pl.ds(j * 32, 16)] = lo * w
                        sb[i, pl.ds(j * 32 + 16, 16)] = hi * w

                pltpu.async_copy(sb, acc_sh.at[dix_v.at[b]], sems_s[b],
                                 add=True)

                @pl.when(c + NRING < nch)
                def _():
                    unpack_src(c + NRING, b)
                    pltpu.async_copy(h_hbm.at[six_v.at[b]], gb, sems_g[b])

            return 0

        lax.fori_loop(0, nch // NRING, ring, 0)
        for b in range(NRING):
            @pl.when(b < nch)
            def _():
                pltpu.make_async_copy(sbuf.at[b], acc_sh.at[dix_v.at[b]],
                                      sems_s[b]).wait()
        pltpu.sync_copy(sloc_v, sp_hbm.at[wid])
        plsc.subcore_barrier()
        sc_ring.__exit__(None, None, None)

        with jax.named_scope("sc_drain"):
            pltpu.sync_copy(acc_sh.at[pl.ds(base, ROWS_PER_TILE)],
                            p_hbm.at[cid, pl.ds(base, ROWS_PER_TILE)])

    return edge_kernel


def _sc64(*args):
    return _sc_edge_phase(D1)(*args)


def kernel(x, edge_index, W1, as1, ad1, b1, W2, as2, ad2, b2,
           W3, as3, ad3, b3, W5, b5, Wo, bo):
    loops = jnp.arange(N, dtype=jnp.int32)
    padi = jnp.full((EPAD - E,), SENT, jnp.int32)
    src = jnp.concatenate([edge_index[0].astype(jnp.int32), loops, padi])
    dst = jnp.concatenate([edge_index[1].astype(jnp.int32), loops, padi])
    flat = (src << 14) | dst
    sentp = (SENT << 14) | SENT
    na = 16 * NCA * CH
    pk_a = flat[:na].reshape(16, NCA, CH)
    pk_b = flat[na:].reshape(16, NCB, CH)
    pk_b = jnp.concatenate(
        [pk_b, jnp.full((16, NCA - NCB, CH), sentp, jnp.int32)], axis=1)
    packed = jnp.concatenate([pk_a, pk_b], axis=0)

    x_pad = jnp.zeros((NPAD, D_IN), jnp.float32).at[:N].set(x)
    wo_pad = jnp.zeros((HDF, 128), jnp.float32).at[:, :D_OUT].set(Wo)
    bo_pad = jnp.zeros((128,), jnp.float32).at[:D_OUT].set(bo)

    b1p, w2p = b1[_P64], W2[_P64, :]
    b2p, w3p = b2[_P128], W3[_P128, :]
    b3p, w5p = b3[_P64], W5[_P64, :]

    h1, asad1, g1 = _tc_first(x_pad, W1, as1, ad1, D1)
    p1, sp1 = _sc64(h1, asad1, g1, packed)
    h2a, h2b, asad2, g2 = _tc_mid(p1, sp1, b1p, w2p, as2, ad2)
    p2a, sp2 = _sc64(h2a, asad2, g2, packed)
    p2b, _sp2b = _sc64(h2b, asad2, g2, packed)
    h3, asad3, g3 = _tc_mid2(p2a, p2b, sp2, b2p, w3p, as3, ad3, D3)
    p3, sp3 = _sc64(h3, asad3, g3, packed)
    out = _tc_fin(p3, sp3, b3p, w5p, b5, wo_pad, bo_pad)
    return out[:N, :D_OUT]


# speedup vs baseline: 2.8082x; 1.0812x over previous
import functools

import jax
import jax.numpy as jnp
from jax import lax
from jax.experimental import pallas as pl
from jax.experimental.pallas import tpu as pltpu
from jax.experimental.pallas import tpu_sc as plsc

N = 10000
E_RAW = 320000
E = E_RAW + N
D_IN = 128
HDF = 16
D1 = 4 * HDF
D2 = 8 * HDF
D3 = 4 * HDF
D_OUT = 4

NPAD = 10240
SENT = N
NW = 32
CH = 128
NRING = 3
NCA = 105
NCB = 57
EPAD = 16 * (NCA + NCB) * CH
ROWS_PER_TILE = NPAD // 16


import numpy as _np


def _perm(dim):
    p = []
    for o in range(0, dim, 32):
        p.extend(o + 2 * k for k in range(16))
        p.extend(o + 2 * k + 1 for k in range(16))
    return _np.array(p)


_P64 = _perm(64)
_P128 = _perm(128)


def _attn_tail(h, ats, atd, h_ref, asad_ref, gmax_ref):
    row2 = lax.broadcasted_iota(jnp.int32, h.shape, 0)
    h = jnp.where(row2 < N, h, 0.0)
    h_ref[...] = h.astype(jnp.bfloat16)
    a_s = jnp.sum(h * ats[None, :], axis=1)
    a_d = jnp.sum(h * atd[None, :], axis=1)
    g = jnp.max(a_s) + jnp.max(a_d)
    g = jnp.where(g > 0, g, 0.2 * g)
    gmax_ref[...] = jnp.full((8, 128), g, jnp.float32)
    ridx = lax.broadcasted_iota(jnp.int32, (2, NPAD), 1)
    asad = jnp.stack([a_s, a_d], axis=0)
    asad_ref[...] = jnp.where(ridx < N, asad, -1e30)


def _tc_first_body(x_ref, w_ref, ats_ref, atd_ref, h_ref, asad_ref, gmax_ref):
    h = jnp.dot(x_ref[...], w_ref[...], preferred_element_type=jnp.float32)
    _attn_tail(h, ats_ref[...], atd_ref[...], h_ref, asad_ref, gmax_ref)


def _tc_mid_body(p_ref, sp_ref, b_ref, w_ref, ats_ref, atd_ref,
                 ha_ref, hb_ref, asad_ref, gmax_ref):
    s = jnp.sum(sp_ref[...], axis=0)
    agg = p_ref[0] + p_ref[1]
    hin = agg / s[:, None] + b_ref[...][None, :]
    hin = hin * jax.nn.sigmoid(hin)
    row2 = lax.broadcasted_iota(jnp.int32, hin.shape, 0)
    hin = jnp.where(row2 < N, hin, 0.0)
    h = jnp.dot(hin, w_ref[...], preferred_element_type=jnp.float32)
    row2 = lax.broadcasted_iota(jnp.int32, h.shape, 0)
    h = jnp.where(row2 < N, h, 0.0)
    ha_ref[...] = h[:, :D1].astype(jnp.bfloat16)
    hb_ref[...] = h[:, D1:].astype(jnp.bfloat16)
    ats, atd = ats_ref[...], atd_ref[...]
    a_s = jnp.sum(h * ats[None, :], axis=1)
    a_d = jnp.sum(h * atd[None, :], axis=1)
    g = jnp.max(a_s) + jnp.max(a_d)
    g = jnp.where(g > 0, g, 0.2 * g)
    gmax_ref[...] = jnp.full((8, 128), g, jnp.float32)
    ridx = lax.broadcasted_iota(jnp.int32, (2, NPAD), 1)
    asad = jnp.stack([a_s, a_d], axis=0)
    asad_ref[...] = jnp.where(ridx < N, asad, -1e30)


def _tc_mid2_body(pa_ref, pb_ref, sp_ref, b_ref, w_ref, ats_ref, atd_ref,
                  h_ref, asad_ref, gmax_ref):
    s = jnp.sum(sp_ref[...], axis=0)
    agg = jnp.concatenate([pa_ref[0] + pa_ref[1], pb_ref[0] + pb_ref[1]],
                          axis=1)
    hin = agg / s[:, None] + b_ref[...][None, :]
    hin = hin * jax.nn.sigmoid(hin)
    row2 = lax.broadcasted_iota(jnp.int32, hin.shape, 0)
    hin = jnp.where(row2 < N, hin, 0.0)
    h = jnp.dot(hin, w_ref[...], preferred_element_type=jnp.float32)
    _attn_tail(h, ats_ref[...], atd_ref[...], h_ref, asad_ref, gmax_ref)


def _tc_fin_body(p_ref, sp_ref, b3_ref, w5_ref, b5_ref, wo_ref, bo_ref,
                 out_ref):
    s = jnp.sum(sp_ref[...], axis=0)
    agg = p_ref[0] + p_ref[1]
    h3 = agg / s[:, None] + b3_ref[...][None, :]
    h3 = h3 * jax.nn.sigmoid(h3)
    row2 = lax.broadcasted_iota(jnp.int32, h3.shape, 0)
    h3 = jnp.where(row2 < N, h3, 0.0)
    h4 = jnp.dot(h3, w5_ref[...], preferred_element_type=jnp.float32)
    h4 = h4 + b5_ref[...][None, :]
    h4 = h4 * jax.nn.sigmoid(h4)
    out = jnp.dot(h4, wo_ref[...], preferred_element_type=jnp.float32)
    out = out + bo_ref[...][None, :]
    out_ref[...] = out * jax.nn.sigmoid(out)


def _tc_first(x_pad, w, ats, atd, dout):
    return pl.pallas_call(
        _tc_first_body,
        out_shape=[
            jax.ShapeDtypeStruct((NPAD, dout), jnp.bfloat16),
            jax.ShapeDtypeStruct((2, NPAD), jnp.float32),
            jax.ShapeDtypeStruct((8, 128), jnp.float32),
        ],
    )(x_pad, w, ats, atd)


def _tc_mid(p, sp, b, w, ats, atd):
    return pl.pallas_call(
        _tc_mid_body,
        out_shape=[
            jax.ShapeDtypeStruct((NPAD, D1), jnp.bfloat16),
            jax.ShapeDtypeStruct((NPAD, D1), jnp.bfloat16),
            jax.ShapeDtypeStruct((2, NPAD), jnp.float32),
            jax.ShapeDtypeStruct((8, 128), jnp.float32),
        ],
    )(p, sp, b, w, ats, atd)


def _tc_mid2(pa, pb, sp, b, w, ats, atd, dout):
    return pl.pallas_call(
        _tc_mid2_body,
        out_shape=[
            jax.ShapeDtypeStruct((NPAD, dout), jnp.bfloat16),
            jax.ShapeDtypeStruct((2, NPAD), jnp.float32),
            jax.ShapeDtypeStruct((8, 128), jnp.float32),
        ],
    )(pa, pb, sp, b, w, ats, atd)


def _tc_fin(p, sp, b3, w5, b5, wo_pad, bo_pad):
    return pl.pallas_call(
        _tc_fin_body,
        out_shape=jax.ShapeDtypeStruct((NPAD, 128), jnp.float32),
    )(p, sp, b3, w5, b5, wo_pad, bo_pad)


@functools.lru_cache(maxsize=None)
def _sc_edge_phase(d):
    mesh = plsc.VectorSubcoreMesh(core_axis_name="c", subcore_axis_name="s",
                                  num_cores=2, num_subcores=16)

    @functools.partial(
        pl.kernel,
        out_type=[
            jax.ShapeDtypeStruct((2, NPAD, d), jnp.float32),
            jax.ShapeDtypeStruct((NW, NPAD), jnp.float32),
        ],
        mesh=mesh,
        compiler_params=pltpu.CompilerParams(needs_layout_passes=False,
                                             use_tc_tiling_on_sc=False),
        scratch_types=[
            pltpu.VMEM((NPAD,), jnp.float32),
            pltpu.VMEM((NPAD,), jnp.float32),
            pltpu.VMEM((NCA, CH), jnp.int32),
            pltpu.VMEM((NPAD,), jnp.float32),
            pltpu.VMEM((CH,), jnp.float32),
            pltpu.VMEM((NRING, CH, d), jnp.bfloat16),
            pltpu.VMEM((NRING, CH, d), jnp.float32),
            pltpu.VMEM((NRING, CH), jnp.int32),
            pltpu.VMEM((NRING, CH), jnp.int32),
            pltpu.VMEM((16,), jnp.float32),
            pltpu.VMEM_SHARED((NPAD, d), jnp.float32),
            pltpu.SemaphoreType.DMA,
            pltpu.SemaphoreType.DMA,
            pltpu.SemaphoreType.DMA,
            pltpu.SemaphoreType.DMA,
            pltpu.SemaphoreType.DMA,
            pltpu.SemaphoreType.DMA,
        ],
    )
    def edge_kernel(h_hbm, asad_hbm, gmax_hbm, pk_hbm,
                    p_hbm, sp_hbm,
                    as_v, ad_v, pk_v, sloc_v, exc_v, gbuf, sbuf, six_v, dix_v,
                    gmax_v, acc_sh, sem_g0, sem_g1, sem_g2,
                    sem_s0, sem_s1, sem_s2):
        cid = lax.axis_index("c")
        sid = lax.axis_index("s")
        wid = cid * 16 + sid
        nch = jnp.where(cid == 0, NCA, NCB)
        sems_g = (sem_g0, sem_g1, sem_g2)
        sems_s = (sem_s0, sem_s1, sem_s2)

        def unpack_src(c, b):
            @plsc.parallel_loop(0, CH // 16, 1, unroll=8)
            def grp(g):
                pk = pk_v[c, pl.ds(g * 16, 16)]
                six_v[b, pl.ds(g * 16, 16)] = lax.shift_right_logical(pk, 14)

        def unpack_dst(c, b):
            @plsc.parallel_loop(0, CH // 16, 1, unroll=8)
            def grp(g):
                pk = pk_v[c, pl.ds(g * 16, 16)]
                dix_v[b, pl.ds(g * 16, 16)] = lax.bitwise_and(pk, 16383)

        pltpu.sync_copy(pk_hbm.at[wid], pk_v)
        for b in range(NRING):
            @pl.when(b < nch)
            def _():
                unpack_src(b, b)
                pltpu.async_copy(h_hbm.at[six_v.at[b]], gbuf.at[b], sems_g[b])
        pltpu.sync_copy(asad_hbm.at[0], as_v)
        pltpu.sync_copy(asad_hbm.at[1], ad_v)
        pltpu.sync_copy(gmax_hbm.at[0, pl.ds(0, 16)], gmax_v)

        zeros16 = jnp.zeros((16,), jnp.float32)

        def zero_rows(i, _):
            for j in range(d // 16):
                sbuf[0, i, pl.ds(j * 16, 16)] = zeros16
            return 0

        lax.fori_loop(0, CH, zero_rows, 0)

        def zero_s(i, _):
            sloc_v[pl.ds(i * 16, 16)] = zeros16
            return 0

        lax.fori_loop(0, NPAD // 16, zero_s, 0)

        base = sid * ROWS_PER_TILE
        sb0 = sbuf.at[0]
        for k in range(ROWS_PER_TILE // CH):
            pltpu.sync_copy(sb0, acc_sh.at[pl.ds(base + k * CH, CH)])
        plsc.subcore_barrier()

        gmax = gmax_v[...]

        sc_ring = jax.named_scope("sc_ring")
        sc_ring.__enter__()

        def ring(cc, _):
            for b in range(NRING):
                c = cc * NRING + b
                gb = gbuf.at[b]
                sb = sbuf.at[b]
                pltpu.make_async_copy(h_hbm.at[six_v.at[b]], gb,
                                      sems_g[b]).wait()

                @pl.when(cc > 0)
                def _():
                    pltpu.make_async_copy(sb, acc_sh.at[dix_v.at[b]],
                                          sems_s[b]).wait()

                unpack_dst(c, b)

                @plsc.parallel_loop(0, CH // 16, 1, unroll=2)
                def grp(g):
                    s16 = six_v[b, pl.ds(g * 16, 16)]
                    d16 = dix_v[b, pl.ds(g * 16, 16)]
                    va = plsc.load_gather(as_v, [s16])
                    vd = plsc.load_gather(ad_v, [d16])
                    e = va + vd
                    e = jnp.where(e > 0, e, 0.2 * e)
                    ex = jnp.exp(e - gmax)
                    exc_v[pl.ds(g * 16, 16)] = ex
                    plsc.addupdate_scatter(sloc_v, [d16], ex)

                @plsc.parallel_loop(0, CH, 1, unroll=4)
                def scale(i):
                    w = plsc.load_gather(exc_v, [jnp.full((16,), i, jnp.int32)])
                    for j in range(d // 32):
                        x = gb[i, pl.ds(j * 32, 32)]
                        lo, hi = plsc.unpack(
                            x, format=plsc.PackFormat.INTERLEAVED)
                        sb[i, pl.ds(j * 32, 16)] = lo * w
                        sb[i, pl.ds(j * 32 + 16, 16)] = hi * w

                pltpu.async_copy(sb, acc_sh.at[dix_v.at[b]], sems_s[b],
                                 add=True)

                @pl.when(c + NRING < nch)
                def _():
                    unpack_src(c + NRING, b)
                    pltpu.async_copy(h_hbm.at[six_v.at[b]], gb, sems_g[b])

            return 0

        lax.fori_loop(0, nch // NRING, ring, 0)
        for b in range(NRING):
            @pl.when(b < nch)
            def _():
                pltpu.make_async_copy(sbuf.at[b], acc_sh.at[dix_v.at[b]],
                                      sems_s[b]).wait()
        pltpu.sync_copy(sloc_v, sp_hbm.at[wid])
        plsc.subcore_barrier()
        sc_ring.__exit__(None, None, None)

        with jax.named_scope("sc_drain"):
            pltpu.sync_copy(acc_sh.at[pl.ds(base, ROWS_PER_TILE)],
                            p_hbm.at[cid, pl.ds(base, ROWS_PER_TILE)])

    return edge_kernel


def _sc64(*args):
    return _sc_edge_phase(D1)(*args)


def kernel(x, edge_index, W1, as1, ad1, b1, W2, as2, ad2, b2,
           W3, as3, ad3, b3, W5, b5, Wo, bo):
    loops = jnp.arange(N, dtype=jnp.int32)
    padi = jnp.full((EPAD - E,), SENT, jnp.int32)
    src = jnp.concatenate([edge_index[0].astype(jnp.int32), loops, padi])
    dst = jnp.concatenate([edge_index[1].astype(jnp.int32), loops, padi])
    flat = (src << 14) | dst
    sentp = (SENT << 14) | SENT
    na = 16 * NCA * CH
    pk_a = flat[:na].reshape(16, NCA, CH)
    pk_b = flat[na:].reshape(16, NCB, CH)
    pk_b = jnp.concatenate(
        [pk_b, jnp.full((16, NCA - NCB, CH), sentp, jnp.int32)], axis=1)
    packed = jnp.concatenate([pk_a, pk_b], axis=0)

    x_pad = jnp.zeros((NPAD, D_IN), jnp.float32).at[:N].set(x)
    wo_pad = jnp.zeros((HDF, 128), jnp.float32).at[:, :D_OUT].set(Wo)
    bo_pad = jnp.zeros((128,), jnp.float32).at[:D_OUT].set(bo)

    b1p, w2p = b1[_P64], W2[_P64, :]
    b2p, w3p = b2[_P128], W3[_P128, :]
    b3p, w5p = b3[_P64], W5[_P64, :]

    h1, asad1, g1 = _tc_first(x_pad, W1, as1, ad1, D1)
    p1, sp1 = _sc64(h1, asad1, g1, packed)
    h2a, h2b, asad2, g2 = _tc_mid(p1, sp1, b1p, w2p, as2, ad2)
    p2a, sp2 = _sc64(h2a, asad2, g2, packed)
    p2b, _sp2b = _sc64(h2b, asad2, g2, packed)
    h3, asad3, g3 = _tc_mid2(p2a, p2b, sp2, b2p, w3p, as3, ad3, D3)
    p3, sp3 = _sc64(h3, asad3, g3, packed)
    out = _tc_fin(p3, sp3, b3p, w5p, b5, wo_pad, bo_pad)
    return out[:N, :D_OUT]


# speedup vs baseline: 2.8825x; 1.0265x over previous
import functools

import jax
import jax.numpy as jnp
from jax import lax
from jax.experimental import pallas as pl
from jax.experimental.pallas import tpu as pltpu
from jax.experimental.pallas import tpu_sc as plsc

N = 10000
E_RAW = 320000
E = E_RAW + N
D_IN = 128
HDF = 16
D1 = 4 * HDF
D2 = 8 * HDF
D3 = 4 * HDF
D_OUT = 4

NPAD = 10240
SENT = N
NW = 32
CH = 128
NRING = 3
NCA = 99
NCB = 63
EPAD = 16 * (NCA + NCB) * CH
ROWS_PER_TILE = NPAD // 16


import numpy as _np


def _perm(dim):
    p = []
    for o in range(0, dim, 32):
        p.extend(o + 2 * k for k in range(16))
        p.extend(o + 2 * k + 1 for k in range(16))
    return _np.array(p)


_P64 = _perm(64)
_P128 = _perm(128)


def _attn_tail(h, ats, atd, h_ref, asad_ref, gmax_ref):
    row2 = lax.broadcasted_iota(jnp.int32, h.shape, 0)
    h = jnp.where(row2 < N, h, 0.0)
    h_ref[...] = h.astype(jnp.bfloat16)
    a_s = jnp.sum(h * ats[None, :], axis=1)
    a_d = jnp.sum(h * atd[None, :], axis=1)
    g = jnp.max(a_s) + jnp.max(a_d)
    g = jnp.where(g > 0, g, 0.2 * g)
    gmax_ref[...] = jnp.full((8, 128), g, jnp.float32)
    ridx = lax.broadcasted_iota(jnp.int32, (2, NPAD), 1)
    asad = jnp.stack([a_s, a_d], axis=0)
    asad_ref[...] = jnp.where(ridx < N, asad, -1e30)


def _tc_first_body(x_ref, w_ref, ats_ref, atd_ref, h_ref, asad_ref, gmax_ref):
    h = jnp.dot(x_ref[...], w_ref[...], preferred_element_type=jnp.float32)
    _attn_tail(h, ats_ref[...], atd_ref[...], h_ref, asad_ref, gmax_ref)


def _tc_mid_body(p_ref, sp_ref, b_ref, w_ref, ats_ref, atd_ref,
                 ha_ref, hb_ref, asad_ref, gmax_ref):
    s = jnp.sum(sp_ref[...], axis=0)
    agg = p_ref[0] + p_ref[1]
    hin = agg / s[:, None] + b_ref[...][None, :]
    hin = hin * jax.nn.sigmoid(hin)
    row2 = lax.broadcasted_iota(jnp.int32, hin.shape, 0)
    hin = jnp.where(row2 < N, hin, 0.0)
    h = jnp.dot(hin, w_ref[...], preferred_element_type=jnp.float32)
    row2 = lax.broadcasted_iota(jnp.int32, h.shape, 0)
    h = jnp.where(row2 < N, h, 0.0)
    ha_ref[...] = h[:, :D1].astype(jnp.bfloat16)
    hb_ref[...] = h[:, D1:].astype(jnp.bfloat16)
    ats, atd = ats_ref[...], atd_ref[...]
    a_s = jnp.sum(h * ats[None, :], axis=1)
    a_d = jnp.sum(h * atd[None, :], axis=1)
    g = jnp.max(a_s) + jnp.max(a_d)
    g = jnp.where(g > 0, g, 0.2 * g)
    gmax_ref[...] = jnp.full((8, 128), g, jnp.float32)
    ridx = lax.broadcasted_iota(jnp.int32, (2, NPAD), 1)
    asad = jnp.stack([a_s, a_d], axis=0)
    asad_ref[...] = jnp.where(ridx < N, asad, -1e30)


def _tc_mid2_body(pa_ref, pb_ref, sp_ref, b_ref, w_ref, ats_ref, atd_ref,
                  h_ref, asad_ref, gmax_ref):
    s = jnp.sum(sp_ref[...], axis=0)
    agg = jnp.concatenate([pa_ref[0] + pa_ref[1], pb_ref[0] + pb_ref[1]],
                          axis=1)
    hin = agg / s[:, None] + b_ref[...][None, :]
    hin = hin * jax.nn.sigmoid(hin)
    row2 = lax.broadcasted_iota(jnp.int32, hin.shape, 0)
    hin = jnp.where(row2 < N, hin, 0.0)
    h = jnp.dot(hin, w_ref[...], preferred_element_type=jnp.float32)
    _attn_tail(h, ats_ref[...], atd_ref[...], h_ref, asad_ref, gmax_ref)


def _tc_fin_body(p_ref, sp_ref, b3_ref, w5_ref, b5_ref, wo_ref, bo_ref,
                 out_ref):
    s = jnp.sum(sp_ref[...], axis=0)
    agg = p_ref[0] + p_ref[1]
    h3 = agg / s[:, None] + b3_ref[...][None, :]
    h3 = h3 * jax.nn.sigmoid(h3)
    row2 = lax.broadcasted_iota(jnp.int32, h3.shape, 0)
    h3 = jnp.where(row2 < N, h3, 0.0)
    h4 = jnp.dot(h3, w5_ref[...], preferred_element_type=jnp.float32)
    h4 = h4 + b5_ref[...][None, :]
    h4 = h4 * jax.nn.sigmoid(h4)
    out = jnp.dot(h4, wo_ref[...], preferred_element_type=jnp.float32)
    out = out + bo_ref[...][None, :]
    out_ref[...] = out * jax.nn.sigmoid(out)


def _tc_first(x_pad, w, ats, atd, dout):
    return pl.pallas_call(
        _tc_first_body,
        out_shape=[
            jax.ShapeDtypeStruct((NPAD, dout), jnp.bfloat16),
            jax.ShapeDtypeStruct((2, NPAD), jnp.float32),
            jax.ShapeDtypeStruct((8, 128), jnp.float32),
        ],
    )(x_pad, w, ats, atd)


def _tc_mid(p, sp, b, w, ats, atd):
    return pl.pallas_call(
        _tc_mid_body,
        out_shape=[
            jax.ShapeDtypeStruct((NPAD, D1), jnp.bfloat16),
            jax.ShapeDtypeStruct((NPAD, D1), jnp.bfloat16),
            jax.ShapeDtypeStruct((2, NPAD), jnp.float32),
            jax.ShapeDtypeStruct((8, 128), jnp.float32),
        ],
    )(p, sp, b, w, ats, atd)


def _tc_mid2(pa, pb, sp, b, w, ats, atd, dout):
    return pl.pallas_call(
        _tc_mid2_body,
        out_shape=[
            jax.ShapeDtypeStruct((NPAD, dout), jnp.bfloat16),
            jax.ShapeDtypeStruct((2, NPAD), jnp.float32),
            jax.ShapeDtypeStruct((8, 128), jnp.float32),
        ],
    )(pa, pb, sp, b, w, ats, atd)


def _tc_fin(p, sp, b3, w5, b5, wo_pad, bo_pad):
    return pl.pallas_call(
        _tc_fin_body,
        out_shape=jax.ShapeDtypeStruct((NPAD, 128), jnp.float32),
    )(p, sp, b3, w5, b5, wo_pad, bo_pad)


@functools.lru_cache(maxsize=None)
def _sc_edge_phase(d):
    mesh = plsc.VectorSubcoreMesh(core_axis_name="c", subcore_axis_name="s",
                                  num_cores=2, num_subcores=16)

    @functools.partial(
        pl.kernel,
        out_type=[
            jax.ShapeDtypeStruct((2, NPAD, d), jnp.float32),
            jax.ShapeDtypeStruct((NW, NPAD), jnp.float32),
        ],
        mesh=mesh,
        compiler_params=pltpu.CompilerParams(needs_layout_passes=False,
                                             use_tc_tiling_on_sc=False),
        scratch_types=[
            pltpu.VMEM((NPAD,), jnp.float32),
            pltpu.VMEM((NPAD,), jnp.float32),
            pltpu.VMEM((NCA, CH), jnp.int32),
            pltpu.VMEM((NPAD,), jnp.float32),
            pltpu.VMEM((CH,), jnp.float32),
            pltpu.VMEM((NRING, CH, d), jnp.bfloat16),
            pltpu.VMEM((NRING, CH, d), jnp.float32),
            pltpu.VMEM((NRING, CH), jnp.int32),
            pltpu.VMEM((NRING, CH), jnp.int32),
            pltpu.VMEM((16,), jnp.float32),
            pltpu.VMEM_SHARED((NPAD, d), jnp.float32),
            pltpu.SemaphoreType.DMA,
            pltpu.SemaphoreType.DMA,
            pltpu.SemaphoreType.DMA,
            pltpu.SemaphoreType.DMA,
            pltpu.SemaphoreType.DMA,
            pltpu.SemaphoreType.DMA,
        ],
    )
    def edge_kernel(h_hbm, asad_hbm, gmax_hbm, pk_hbm,
                    p_hbm, sp_hbm,
                    as_v, ad_v, pk_v, sloc_v, exc_v, gbuf, sbuf, six_v, dix_v,
                    gmax_v, acc_sh, sem_g0, sem_g1, sem_g2,
                    sem_s0, sem_s1, sem_s2):
        cid = lax.axis_index("c")
        sid = lax.axis_index("s")
        wid = cid * 16 + sid
        nch = jnp.where(cid == 0, NCA, NCB)
        sems_g = (sem_g0, sem_g1, sem_g2)
        sems_s = (sem_s0, sem_s1, sem_s2)

        def unpack_src(c, b):
            @plsc.parallel_loop(0, CH // 16, 1, unroll=8)
            def grp(g):
                pk = pk_v[c, pl.ds(g * 16, 16)]
                six_v[b, pl.ds(g * 16, 16)] = lax.shift_right_logical(pk, 14)

        def unpack_dst(c, b):
            @plsc.parallel_loop(0, CH // 16, 1, unroll=8)
            def grp(g):
                pk = pk_v[c, pl.ds(g * 16, 16)]
                dix_v[b, pl.ds(g * 16, 16)] = lax.bitwise_and(pk, 16383)

        pltpu.sync_copy(pk_hbm.at[wid], pk_v)
        for b in range(NRING):
            @pl.when(b < nch)
            def _():
                unpack_src(b, b)
                pltpu.async_copy(h_hbm.at[six_v.at[b]], gbuf.at[b], sems_g[b])
        pltpu.sync_copy(asad_hbm.at[0], as_v)
        pltpu.sync_copy(asad_hbm.at[1], ad_v)
        pltpu.sync_copy(gmax_hbm.at[0, pl.ds(0, 16)], gmax_v)

        zeros16 = jnp.zeros((16,), jnp.float32)

        def zero_rows(i, _):
            for j in range(d // 16):
                sbuf[0, i, pl.ds(j * 16, 16)] = zeros16
            return 0

        lax.fori_loop(0, CH, zero_rows, 0)

        def zero_s(i, _):
            sloc_v[pl.ds(i * 16, 16)] = zeros16
            return 0

        lax.fori_loop(0, NPAD // 16, zero_s, 0)

        base = sid * ROWS_PER_TILE
        sb0 = sbuf.at[0]
        for k in range(ROWS_PER_TILE // CH):
            pltpu.sync_copy(sb0, acc_sh.at[pl.ds(base + k * CH, CH)])
        plsc.subcore_barrier()

        gmax = gmax_v[...]

        sc_ring = jax.named_scope("sc_ring")
        sc_ring.__enter__()

        def ring(cc, _):
            for b in range(NRING):
                c = cc * NRING + b
                gb = gbuf.at[b]
                sb = sbuf.at[b]
                pltpu.make_async_copy(h_hbm.at[six_v.at[b]], gb,
                                      sems_g[b]).wait()

                @pl.when(cc > 0)
                def _():
                    pltpu.make_async_copy(sb, acc_sh.at[dix_v.at[b]],
                                          sems_s[b]).wait()

                unpack_dst(c, b)

                @plsc.parallel_loop(0, CH // 16, 1, unroll=2)
                def grp(g):
                    s16 = six_v[b, pl.ds(g * 16, 16)]
                    d16 = dix_v[b, pl.ds(g * 16, 16)]
                    va = plsc.load_gather(as_v, [s16])
                    vd = plsc.load_gather(ad_v, [d16])
                    e = va + vd
                    e = jnp.where(e > 0, e, 0.2 * e)
                    ex = jnp.exp(e - gmax)
                    exc_v[pl.ds(g * 16, 16)] = ex
                    plsc.addupdate_scatter(sloc_v, [d16], ex)

                @plsc.parallel_loop(0, CH, 1, unroll=4)
                def scale(i):
                    w = plsc.load_gather(exc_v, [jnp.full((16,), i, jnp.int32)])
                    for j in range(d // 32):
                        x = gb[i, pl.ds(j * 32, 32)]
                        lo, hi = plsc.unpack(
                            x, format=plsc.PackFormat.INTERLEAVED)
                        sb[i, pl.ds(j * 32, 16)] = lo * w
                        sb[i, pl.ds(j * 32 + 16, 16)] = hi * w

                pltpu.async_copy(sb, acc_sh.at[dix_v.at[b]], sems_s[b],
                                 add=True)

                @pl.when(c + NRING < nch)
                def _():
                    unpack_src(c + NRING, b)
                    pltpu.async_copy(h_hbm.at[six_v.at[b]], gb, sems_g[b])

            return 0

        lax.fori_loop(0, nch // NRING, ring, 0)
        for b in range(NRING):
            @pl.when(b < nch)
            def _():
                pltpu.make_async_copy(sbuf.at[b], acc_sh.at[dix_v.at[b]],
                                      sems_s[b]).wait()
        pltpu.sync_copy(sloc_v, sp_hbm.at[wid])
        plsc.subcore_barrier()
        sc_ring.__exit__(None, None, None)

        with jax.named_scope("sc_drain"):
            pltpu.sync_copy(acc_sh.at[pl.ds(base, ROWS_PER_TILE)],
                            p_hbm.at[cid, pl.ds(base, ROWS_PER_TILE)])

    return edge_kernel


def _sc64(*args):
    return _sc_edge_phase(D1)(*args)


def kernel(x, edge_index, W1, as1, ad1, b1, W2, as2, ad2, b2,
           W3, as3, ad3, b3, W5, b5, Wo, bo):
    loops = jnp.arange(N, dtype=jnp.int32)
    padi = jnp.full((EPAD - E,), SENT, jnp.int32)
    src = jnp.concatenate([edge_index[0].astype(jnp.int32), loops, padi])
    dst = jnp.concatenate([edge_index[1].astype(jnp.int32), loops, padi])
    flat = (src << 14) | dst
    sentp = (SENT << 14) | SENT
    na = 16 * NCA * CH
    pk_a = flat[:na].reshape(16, NCA, CH)
    pk_b = flat[na:].reshape(16, NCB, CH)
    pk_b = jnp.concatenate(
        [pk_b, jnp.full((16, NCA - NCB, CH), sentp, jnp.int32)], axis=1)
    packed = jnp.concatenate([pk_a, pk_b], axis=0)

    x_pad = jnp.zeros((NPAD, D_IN), jnp.float32).at[:N].set(x)
    wo_pad = jnp.zeros((HDF, 128), jnp.float32).at[:, :D_OUT].set(Wo)
    bo_pad = jnp.zeros((128,), jnp.float32).at[:D_OUT].set(bo)

    b1p, w2p = b1[_P64], W2[_P64, :]
    b2p, w3p = b2[_P128], W3[_P128, :]
    b3p, w5p = b3[_P64], W5[_P64, :]

    h1, asad1, g1 = _tc_first(x_pad, W1, as1, ad1, D1)
    p1, sp1 = _sc64(h1, asad1, g1, packed)
    h2a, h2b, asad2, g2 = _tc_mid(p1, sp1, b1p, w2p, as2, ad2)
    p2a, sp2 = _sc64(h2a, asad2, g2, packed)
    p2b, _sp2b = _sc64(h2b, asad2, g2, packed)
    h3, asad3, g3 = _tc_mid2(p2a, p2b, sp2, b2p, w3p, as3, ad3, D3)
    p3, sp3 = _sc64(h3, asad3, g3, packed)
    out = _tc_fin(p3, sp3, b3p, w5p, b5, wo_pad, bo_pad)
    return out[:N, :D_OUT]
